# baseline (speedup 1.0000x reference)
import functools

import jax
import jax.numpy as jnp
from jax import lax
from jax.experimental import pallas as pl
from jax.experimental.pallas import tpu as pltpu
from jax.experimental.pallas import tpu_sc as plsc

NROW = 1000000
MAIN = 999936
H1 = 499712
H2 = 500224
SH = 31232
SH15 = 31744
CHUNK = 16


def _sc_body(B, K, D, W, pairs_hbm, obs_hbm, samp_hbm, tabT, tail_hbm,
             rel_tab, predT_out, pos_out, neg_out,
             pairs_i, obs_i, samp_i, idxc, pv, pvd, tailv, A, S,
             pos_s, neg_s, slabA, slabB, semp, semr, semg):
  nc = 2
  c = lax.axis_index("c")
  sid = lax.axis_index("s")
  base = sid * W
  d0 = pl.multiple_of(c * (D // 2), D // 2)
  rows16 = lax.iota(jnp.int32, 16)
  nd = D // 2

  pltpu.sync_copy(pairs_hbm.at[pl.ds(base, W)], pairs_i)
  pltpu.sync_copy(obs_hbm.at[pl.ds(base, W)], obs_i)
  for k in range(K):
    pltpu.sync_copy(samp_hbm.at[pl.ds(k * B + base, W)], samp_i.at[k])
  pltpu.sync_copy(tail_hbm.at[pl.ds(d0, nd), :], tailv)

  def load_slab(dd, h, slab):
    if h == 0:
      pltpu.async_copy(tabT.at[d0 + dd, pl.ds(sid * SH, SH)],
                       slab.at[pl.ds(sid * SH, SH)], semp)
    else:
      @pl.when(sid < 15)
      def _():
        pltpu.async_copy(tabT.at[d0 + dd, pl.ds(H1 + sid * SH, SH)],
                         slab.at[pl.ds(sid * SH, SH)], semp)
      @pl.when(sid == 15)
      def _():
        pltpu.async_copy(tabT.at[d0 + dd, pl.ds(H1 + 15 * SH, SH15)],
                         slab.at[pl.ds(15 * SH, SH15)], semp)

  def drain_slab(h, slab):
    if h == 0:
      pltpu.make_async_copy(tabT.at[0, pl.ds(0, SH)],
                            slab.at[pl.ds(0, SH)], semp).wait()
    else:
      @pl.when(sid < 15)
      def _():
        pltpu.make_async_copy(tabT.at[0, pl.ds(0, SH)],
                              slab.at[pl.ds(0, SH)], semp).wait()
      @pl.when(sid == 15)
      def _():
        pltpu.make_async_copy(tabT.at[0, pl.ds(0, SH15)],
                              slab.at[pl.ds(0, SH15)], semp).wait()

  def use_slab(dd, h, slab):
    dsplat = jnp.zeros((16,), jnp.int32) + dd
    lo = 0 if h == 0 else H1
    sz = H1 if h == 0 else H2
    for j in range(W // 128):
      for g2 in range(8):
        v = pairs_i[pl.ds(j * 128 + g2 * 16, 16)]
        idxc[j, pl.ds(g2 * 16, 16)] = jnp.clip(v - lo, 0, sz - 1)
      pltpu.async_copy(slab.at[idxc.at[j]],
                       pvd.at[pl.ds(j * 128, 128)], semg)
    for j in range(W // 128):
      pltpu.make_async_copy(slab.at[idxc.at[0]],
                            pvd.at[pl.ds(0, 128)], semg).wait()
    for j in range(W // 128):
      for g2 in range(8):
        off = j * 128 + g2 * 16
        pos16 = rows16 + off
        v = pairs_i[pl.ds(off, 16)]
        got = pvd[pl.ds(off, 16)]
        inh = (v >= lo) & (v < lo + sz)
        if h == 0:
          val = got
          m = inh
        else:
          mt = v >= MAIN
          tloc = jnp.clip(v - MAIN, 0, tailv.shape[1] - 1)
          tval = plsc.load_gather(tailv, [dsplat, tloc])
          val = jnp.where(mt, tval, got)
          m = inh | mt
        old = plsc.load_gather(pv, [dsplat, pos16])
        plsc.store_scatter(pv, [dsplat, pos16], jnp.where(m, val, old))

  load_slab(0, 0, slabA)
  drain_slab(0, slabA)
  plsc.subcore_barrier()

  def d_body(i, carry):
    load_slab(i, 1, slabB)
    use_slab(i, 0, slabA)
    drain_slab(1, slabB)
    plsc.subcore_barrier()
    @pl.when(i < nd - 1)
    def _():
      load_slab(i + 1, 0, slabA)
    use_slab(i, 1, slabB)
    @pl.when(i < nd - 1)
    def _():
      drain_slab(0, slabA)
    plsc.subcore_barrier()
    return carry

  lax.fori_loop(0, nd, d_body, 0)

  for dg in range(nd // 8):
    for k in range(K):
      pltpu.sync_copy(
          pv.at[pl.ds(dg * 8, 8)],
          predT_out.at[pl.ds(d0 + dg * 8, 8), pl.ds(k * B + base, W)])

  n_chunks = W // CHUNK

  for ch in range(n_chunks):
    rbase = ch * CHUNK
    pltpu.async_copy(rel_tab.at[obs_i.at[pl.ds(rbase, CHUNK)]], A, semr)
    for k in range(K):
      pltpu.async_copy(rel_tab.at[samp_i.at[k, pl.ds(rbase, CHUNK)]],
                       S.at[pl.ds(k * CHUNK, CHUNK)], semr)
    pltpu.make_async_copy(rel_tab.at[pl.ds(0, CHUNK)], A, semr).wait()
    pltpu.make_async_copy(rel_tab.at[pl.ds(0, K * CHUNK)], S, semr).wait()

    def g_body(g, carry):
      row_r = rows16 + g * 16
      pos16 = row_r + rbase

      def dd_body(dd, accs):
        dsplat = jnp.zeros((16,), jnp.int32) + dd
        col = dsplat + d0
        pcol = plsc.load_gather(pv, [dsplat, pos16])
        ocol = plsc.load_gather(A, [row_r, col])
        acc_p = accs[0] + pcol * ocol
        acc_n = [accs[1 + k] +
                 pcol * plsc.load_gather(S, [row_r + k * CHUNK, col])
                 for k in range(K)]
        return tuple([acc_p] + acc_n)

      zero = jnp.zeros((16,), jnp.float32)
      accs = lax.fori_loop(0, nd, dd_body,
                           tuple(zero for _ in range(K + 1)))
      pos_s[pl.ds(rbase + g * 16, 16)] = accs[0]
      for k in range(K):
        neg_s[k, pl.ds(rbase + g * 16, 16)] = accs[1 + k]
      return carry

    lax.fori_loop(0, CHUNK // 16, g_body, 0)

  pltpu.sync_copy(pos_s, pos_out.at[pl.ds(c * B + base, W)])
  for k in range(K):
    pltpu.sync_copy(neg_s.at[k],
                    neg_out.at[pl.ds(c * K * B + k * B + base, W)])


def _sc_call(pairs, obs, samp, pair_tabT, tail, rel_pad):
  B = pairs.shape[0]
  K = samp.shape[0] // B
  D = pair_tabT.shape[0]
  W = B // 16
  mesh = plsc.VectorSubcoreMesh(core_axis_name="c", subcore_axis_name="s")
  body = functools.partial(_sc_body, B, K, D, W)
  f = pl.kernel(
      body,
      out_type=[
          jax.ShapeDtypeStruct((D, K * B), jnp.float32),
          jax.ShapeDtypeStruct((2 * B,), jnp.float32),
          jax.ShapeDtypeStruct((2 * K * B,), jnp.float32),
      ],
      mesh=mesh,
      compiler_params=pltpu.CompilerParams(
          needs_layout_passes=False,
          use_tc_tiling_on_sc=True,
      ),
      scratch_types=[
          pltpu.VMEM((W,), jnp.int32),
          pltpu.VMEM((W,), jnp.int32),
          pltpu.VMEM((K, W), jnp.int32),
          pltpu.VMEM((W // 128, 128), jnp.int32),
          pltpu.VMEM((D // 2, W), jnp.float32),
          pltpu.VMEM((W,), jnp.float32),
          pltpu.VMEM((D // 2, 128), jnp.float32),
          pltpu.VMEM((CHUNK, 128), jnp.float32),
          pltpu.VMEM((K * CHUNK, 128), jnp.float32),
          pltpu.VMEM((W,), jnp.float32),
          pltpu.VMEM((K, W), jnp.float32),
          pltpu.VMEM_SHARED((H1,), jnp.float32),
          pltpu.VMEM_SHARED((H2,), jnp.float32),
          pltpu.SemaphoreType.DMA,
          pltpu.SemaphoreType.DMA,
          pltpu.SemaphoreType.DMA,
      ],
  )
  predT, pos2, neg2 = f(pairs, obs, samp, pair_tabT, tail, rel_pad)
  return predT.T, pos2.reshape(2, B), neg2.reshape(2, K * B)


def _tc_body(pos_ref, neg_ref, obsp_ref, sampp_ref, loss_ref, pl_ref, nl_ref):
  pos = pos_ref[0, :] + pos_ref[1, :]
  neg = neg_ref[0, :] + neg_ref[1, :]
  obsp_ref[...] = jax.nn.sigmoid(pos)
  sampp_ref[...] = jax.nn.sigmoid(neg)
  pos_ls = jnp.minimum(pos, 0.0) - jnp.log1p(jnp.exp(-jnp.abs(pos)))
  neg_ls = jnp.minimum(-neg, 0.0) - jnp.log1p(jnp.exp(-jnp.abs(neg)))
  p_loss = -jnp.sum(pos_ls)
  n_loss = -jnp.sum(neg_ls)
  pl_ref[0, 0] = p_loss
  nl_ref[0, 0] = n_loss
  loss_ref[0, 0] = p_loss + n_loss


def _tc_call(pos2, neg2):
  B = pos2.shape[1]
  KB = neg2.shape[1]
  smem = pl.BlockSpec(memory_space=pltpu.SMEM)
  obsp, sampp, loss, pl_, nl = pl.pallas_call(
      _tc_body,
      out_shape=[
          jax.ShapeDtypeStruct((B,), jnp.float32),
          jax.ShapeDtypeStruct((KB,), jnp.float32),
          jax.ShapeDtypeStruct((1, 1), jnp.float32),
          jax.ShapeDtypeStruct((1, 1), jnp.float32),
          jax.ShapeDtypeStruct((1, 1), jnp.float32),
      ],
      out_specs=[
          pl.BlockSpec(memory_space=pltpu.VMEM),
          pl.BlockSpec(memory_space=pltpu.VMEM),
          smem, smem, smem,
      ],
  )(pos2, neg2)
  return obsp, sampp, loss[0, 0], pl_[0, 0], nl[0, 0]


def kernel(pairs, observed_relations, sampled_relations, pair_table, rel_table):
  pairs = pairs.astype(jnp.int32)
  obs = observed_relations.reshape(-1).astype(jnp.int32)
  samp = sampled_relations.reshape(-1).astype(jnp.int32)
  pair_tabT = pair_table.T
  tail = jnp.pad(pair_table[MAIN:].T, ((0, 0), (0, 128 - (NROW - MAIN))))
  rel_pad = jnp.pad(rel_table, ((0, 0), (0, 128 - rel_table.shape[1])))
  pred_rep, pos2, neg2 = _sc_call(pairs, obs, samp, pair_tabT, tail, rel_pad)
  obs_p, samp_p, loss, p_loss, n_loss = _tc_call(pos2, neg2)
  return (pred_rep, loss, p_loss, n_loss, obs_p, samp_p)


# speedup vs baseline: 1.1306x; 1.1306x over previous
import functools

import jax
import jax.numpy as jnp
from jax import lax
from jax.experimental import pallas as pl
from jax.experimental.pallas import tpu as pltpu
from jax.experimental.pallas import tpu_sc as plsc

NUM_TILES = 32
CHUNK = 128


def _sc_body(B, K, D, W, pairs_hbm, obs_hbm, samp_hbm, pair_tab, rel_tab,
             predT_out, pos_out, neg_out,
             idx_p, idx_o, idx_s, P0, P1, PT, A, S, pos_s, neg_s,
             semp, semr):
  nc = 2
  wid = lax.axis_index("s") * nc + lax.axis_index("c")
  base = wid * W
  rows16 = lax.iota(jnp.int32, 16)
  n_chunks = W // CHUNK
  n_groups = CHUNK // 16
  pbufs = [P0, P1]

  def stage_idx(ch):
    rbase = base + ch * CHUNK
    pltpu.sync_copy(pairs_hbm.at[pl.ds(rbase, CHUNK)], idx_p.at[ch])
    pltpu.sync_copy(obs_hbm.at[pl.ds(rbase, CHUNK)], idx_o.at[ch])
    for k in range(K):
      pltpu.sync_copy(samp_hbm.at[pl.ds(k * B + rbase, CHUNK)],
                      idx_s.at[ch, k])

  def issue_pair(ch):
    P = pbufs[ch % 2]

    def g_issue(g, carry):
      v = idx_p[ch, pl.ds(g * 16, 16)]
      for j in range(16):
        pltpu.async_copy(pair_tab.at[pl.ds(v[j], 1)],
                         P.at[pl.ds(g * 16 + j, 1)], semp)
      return carry

    lax.fori_loop(0, n_groups, g_issue, 0)

  def drain_pair(ch):
    pltpu.make_async_copy(pair_tab.at[pl.ds(0, CHUNK)], pbufs[ch % 2],
                          semp).wait()

  def issue_rel(ch):
    pltpu.async_copy(rel_tab.at[idx_o.at[ch]], A, semr)
    for k in range(K):
      pltpu.async_copy(rel_tab.at[idx_s.at[ch, k]],
                       S.at[pl.ds(k * CHUNK, CHUNK)], semr)

  def drain_rel():
    pltpu.make_async_copy(rel_tab.at[pl.ds(0, CHUNK)], A, semr).wait()
    pltpu.make_async_copy(rel_tab.at[pl.ds(0, K * CHUNK)], S, semr).wait()

  def compute(ch):
    P = pbufs[ch % 2]

    def g_body(g, carry):
      row_r = rows16 + g * 16

      def d_body(dd, accs):
        acc_p = accs[0]
        acc_n = list(accs[1:])
        for u in range(4):
          d = dd * 4 + u
          col = jnp.zeros((16,), jnp.int32) + d
          pcol = plsc.load_gather(P, [row_r, col])
          plsc.store_scatter(PT, [col, row_r], pcol)
          ocol = plsc.load_gather(A, [row_r, col])
          acc_p = acc_p + pcol * ocol
          for k in range(K):
            scol = plsc.load_gather(S, [row_r + k * CHUNK, col])
            acc_n[k] = acc_n[k] + pcol * scol
        return tuple([acc_p] + acc_n)

      zero = jnp.zeros((16,), jnp.float32)
      accs = lax.fori_loop(0, D // 4, d_body,
                           tuple(zero for _ in range(K + 1)))
      off = ch * CHUNK + g * 16
      pos_s[pl.ds(off, 16)] = accs[0]
      for k in range(K):
        neg_s[k, pl.ds(off, 16)] = accs[1 + k]
      return carry

    lax.fori_loop(0, n_groups, g_body, 0)
    rbase = base + ch * CHUNK
    for k in range(K):
      pltpu.sync_copy(PT, predT_out.at[:, pl.ds(k * B + rbase, CHUNK)])

  for ch in range(n_chunks):
    stage_idx(ch)
  issue_pair(0)
  for ch in range(n_chunks):
    issue_rel(ch)
    drain_pair(ch)
    if ch + 1 < n_chunks:
      issue_pair(ch + 1)
    drain_rel()
    compute(ch)

  pltpu.sync_copy(pos_s, pos_out.at[pl.ds(base, W)])
  for k in range(K):
    pltpu.sync_copy(neg_s.at[k], neg_out.at[pl.ds(k * B + base, W)])


def _sc_call(pairs, obs, samp, pair_table, rel_pad):
  B = pairs.shape[0]
  K = samp.shape[0] // B
  D = pair_table.shape[1]
  W = B // NUM_TILES
  n_chunks = W // CHUNK
  mesh = plsc.VectorSubcoreMesh(core_axis_name="c", subcore_axis_name="s")
  body = functools.partial(_sc_body, B, K, D, W)
  f = pl.kernel(
      body,
      out_type=[
          jax.ShapeDtypeStruct((D, K * B), jnp.float32),
          jax.ShapeDtypeStruct((B,), jnp.float32),
          jax.ShapeDtypeStruct((K * B,), jnp.float32),
      ],
      mesh=mesh,
      compiler_params=pltpu.CompilerParams(
          needs_layout_passes=False,
          use_tc_tiling_on_sc=True,
      ),
      scratch_types=[
          pltpu.VMEM((n_chunks, CHUNK), jnp.int32),
          pltpu.VMEM((n_chunks, CHUNK), jnp.int32),
          pltpu.VMEM((n_chunks, K, CHUNK), jnp.int32),
          pltpu.VMEM((CHUNK, D), jnp.float32),
          pltpu.VMEM((CHUNK, D), jnp.float32),
          pltpu.VMEM((D, CHUNK), jnp.float32),
          pltpu.VMEM((CHUNK, 2 * D), jnp.float32),
          pltpu.VMEM((K * CHUNK, 2 * D), jnp.float32),
          pltpu.VMEM((W,), jnp.float32),
          pltpu.VMEM((K, W), jnp.float32),
          pltpu.SemaphoreType.DMA,
          pltpu.SemaphoreType.DMA,
      ],
  )
  predT, pos, neg = f(pairs, obs, samp, pair_table, rel_pad)
  return predT.T, pos, neg


def _tc_body(pos_ref, neg_ref, obsp_ref, sampp_ref, loss_ref, pl_ref, nl_ref):
  pos = pos_ref[...]
  neg = neg_ref[...]
  obsp_ref[...] = jax.nn.sigmoid(pos)
  sampp_ref[...] = jax.nn.sigmoid(neg)
  pos_ls = jnp.minimum(pos, 0.0) - jnp.log1p(jnp.exp(-jnp.abs(pos)))
  neg_ls = jnp.minimum(-neg, 0.0) - jnp.log1p(jnp.exp(-jnp.abs(neg)))
  p_loss = -jnp.sum(pos_ls)
  n_loss = -jnp.sum(neg_ls)
  pl_ref[0, 0] = p_loss
  nl_ref[0, 0] = n_loss
  loss_ref[0, 0] = p_loss + n_loss


def _tc_call(pos_scores, neg_scores):
  B = pos_scores.shape[0]
  KB = neg_scores.shape[0]
  smem = pl.BlockSpec(memory_space=pltpu.SMEM)
  obsp, sampp, loss, pl_, nl = pl.pallas_call(
      _tc_body,
      out_shape=[
          jax.ShapeDtypeStruct((B,), jnp.float32),
          jax.ShapeDtypeStruct((KB,), jnp.float32),
          jax.ShapeDtypeStruct((1, 1), jnp.float32),
          jax.ShapeDtypeStruct((1, 1), jnp.float32),
          jax.ShapeDtypeStruct((1, 1), jnp.float32),
      ],
      out_specs=[
          pl.BlockSpec(memory_space=pltpu.VMEM),
          pl.BlockSpec(memory_space=pltpu.VMEM),
          smem, smem, smem,
      ],
  )(pos_scores, neg_scores)
  return obsp, sampp, loss[0, 0], pl_[0, 0], nl[0, 0]


def kernel(pairs, observed_relations, sampled_relations, pair_table, rel_table):
  pairs = pairs.astype(jnp.int32)
  obs = observed_relations.reshape(-1).astype(jnp.int32)
  samp = sampled_relations.reshape(-1).astype(jnp.int32)
  rel_pad = jnp.pad(rel_table, ((0, 0), (0, 128 - rel_table.shape[1])))
  pred_rep, pos_scores, neg_scores = _sc_call(
      pairs, obs, samp, pair_table, rel_pad)
  obs_p, samp_p, loss, p_loss, n_loss = _tc_call(pos_scores, neg_scores)
  return (pred_rep, loss, p_loss, n_loss, obs_p, samp_p)
